# Initial kernel scaffold; baseline (speedup 1.0000x reference)
import jax
import jax.numpy as jnp
from jax.experimental import pallas as pl


def kernel(x, edge_index, W0, b0, g0, be0, W1, b1, g1, be1, W2, b2, g2, be2, W3, b3, g3, be3, hW0, hb0, hW1, hb1):
    raise NotImplementedError("write your pallas kernel here")


# speedup vs baseline: 10.7245x; 10.7245x over previous
import functools

import jax
import jax.numpy as jnp
from jax import lax
from jax.experimental import pallas as pl
from jax.experimental.pallas import tpu as pltpu
from jax.experimental.pallas import tpu_sc as plsc

N = 10000
E = 320000
HID = 64
EPS = 1e-5

NC = 2
NS = 16
NW = NC * NS
NP = 10240
EP = 327680
CH = 128
NCH = EP // (NW * CH)
RPT = NP // NS

_mesh = plsc.VectorSubcoreMesh(
    core_axis_name="c", subcore_axis_name="s", num_cores=NC, num_subcores=NS)


@functools.partial(
    pl.kernel,
    out_type=jax.ShapeDtypeStruct((NC * NP, 16), jnp.float32),
    mesh=_mesh,
    scratch_types=[
        pltpu.VMEM((NCH, CH), jnp.int32),
        pltpu.VMEM((CH, 16), jnp.float32),
        pltpu.VMEM((RPT, 16), jnp.float32),
        pltpu.VMEM_SHARED((NP, 16), jnp.float32),
    ],
    compiler_params=pltpu.CompilerParams(use_tc_tiling_on_sc=False),
)
def _deg_kernel(dst_hbm, ones_hbm, zeros_hbm, out_hbm, dst_v, ones_v, stage_v, acc_s):
    c = lax.axis_index("c")
    s = lax.axis_index("s")
    wid = c * NS + s
    r0 = s * RPT
    pltpu.sync_copy(zeros_hbm.at[pl.ds(r0, RPT)], stage_v)
    pltpu.sync_copy(stage_v, acc_s.at[pl.ds(r0, RPT)])
    pltpu.sync_copy(ones_hbm, ones_v)
    pltpu.sync_copy(dst_hbm.at[pl.ds(wid * NCH, NCH)], dst_v)
    plsc.subcore_barrier()

    def body(j, carry):
        pltpu.sync_copy(ones_v, acc_s.at[dst_v.at[j]], add=True)
        return carry

    lax.fori_loop(0, NCH, body, 0)
    plsc.subcore_barrier()
    pltpu.sync_copy(acc_s.at[pl.ds(r0, RPT)], stage_v)
    pltpu.sync_copy(stage_v, out_hbm.at[pl.ds(c * NP + r0, RPT)])


@functools.partial(
    pl.kernel,
    out_type=jax.ShapeDtypeStruct((NC * NP, HID), jnp.float32),
    mesh=_mesh,
    scratch_types=[
        pltpu.VMEM((NCH, CH), jnp.int32),
        pltpu.VMEM((NCH, CH), jnp.int32),
        pltpu.VMEM((CH, HID), jnp.float32),
        pltpu.VMEM((RPT, HID), jnp.float32),
        pltpu.VMEM_SHARED((NP, HID), jnp.float32),
        pltpu.SemaphoreType.DMA,
    ],
    compiler_params=pltpu.CompilerParams(use_tc_tiling_on_sc=False),
)
def _scatter_kernel(t_hbm, src_hbm, dst_hbm, out_hbm, src_v, dst_v, rows_v, stage_v, acc_s, sem):
    c = lax.axis_index("c")
    s = lax.axis_index("s")
    wid = c * NS + s
    r0 = s * RPT
    pltpu.sync_copy(t_hbm.at[pl.ds(r0, RPT)], stage_v)
    pltpu.sync_copy(stage_v, acc_s.at[pl.ds(r0, RPT)])
    pltpu.sync_copy(src_hbm.at[pl.ds(wid * NCH, NCH)], src_v)
    pltpu.sync_copy(dst_hbm.at[pl.ds(wid * NCH, NCH)], dst_v)
    plsc.subcore_barrier()

    def body(j, carry):
        pltpu.async_copy(t_hbm.at[src_v.at[j]], rows_v, sem).wait()
        pltpu.sync_copy(rows_v, acc_s.at[dst_v.at[j]], add=True)
        return carry

    lax.fori_loop(0, NCH, body, 0)
    plsc.subcore_barrier()
    pltpu.sync_copy(acc_s.at[pl.ds(r0, RPT)], stage_v)
    pltpu.sync_copy(stage_v, out_hbm.at[pl.ds(c * NP + r0, RPT)])


def _dinv(degp_ref):
    deg = degp_ref[:N, :1] + degp_ref[NP:NP + N, :1] + 1.0
    return lax.rsqrt(deg)


def _pre_body(x_ref, w_ref, degp_ref, t_ref):
    dinv = _dinv(degp_ref)
    t_ref[:N, :] = jnp.dot(
        x_ref[...], w_ref[...], preferred_element_type=jnp.float32) * dinv
    t_ref[N:, :] = jnp.zeros((NP - N, HID), jnp.float32)


def _bn_relu(s_ref, t_ref, degp_ref, b_ref, g_ref, be_ref):
    dinv = _dinv(degp_ref)
    z = (s_ref[:N, :] + s_ref[NP:NP + N, :] - t_ref[:N, :]) * dinv + b_ref[...]
    mu = jnp.mean(z, axis=0, keepdims=True)
    d = z - mu
    var = jnp.mean(d * d, axis=0, keepdims=True)
    hn = jnp.maximum(d * lax.rsqrt(var + EPS) * g_ref[...] + be_ref[...], 0.0)
    return hn, dinv


def _layer_body(s_ref, t_ref, degp_ref, b_ref, g_ref, be_ref, w_ref, o_ref):
    hn, dinv = _bn_relu(s_ref, t_ref, degp_ref, b_ref, g_ref, be_ref)
    o_ref[:N, :] = jnp.dot(
        hn, w_ref[...], preferred_element_type=jnp.float32) * dinv
    o_ref[N:, :] = jnp.zeros((NP - N, HID), jnp.float32)


def _final_body(s_ref, t_ref, degp_ref, b_ref, g_ref, be_ref,
                hw0_ref, hb0_ref, hw1_ref, hb1_ref, o_ref):
    hn, _ = _bn_relu(s_ref, t_ref, degp_ref, b_ref, g_ref, be_ref)
    h2 = jnp.maximum(
        jnp.dot(hn, hw0_ref[...], preferred_element_type=jnp.float32) + hb0_ref[...], 0.0)
    o_ref[...] = jnp.sum(h2 * hw1_ref[...], axis=1, keepdims=True) + hb1_ref[...]


def kernel(x, edge_index, W0, b0, g0, be0, W1, b1, g1, be1, W2, b2, g2, be2,
           W3, b3, g3, be3, hW0, hb0, hW1, hb1):
    pad = jnp.full((EP - E,), N, dtype=jnp.int32)
    src = jnp.concatenate([edge_index[0], pad]).reshape(NW * NCH, CH)
    dst = jnp.concatenate([edge_index[1], pad]).reshape(NW * NCH, CH)
    ones16 = jnp.ones((CH, 16), jnp.float32)
    zeros16 = jnp.zeros((NP, 16), jnp.float32)

    degp = _deg_kernel(dst, ones16, zeros16)

    t = pl.pallas_call(
        _pre_body,
        out_shape=jax.ShapeDtypeStruct((NP, HID), jnp.float32),
    )(x, W0, degp)

    bs = [b0, b1, b2, b3]
    gs = [g0, g1, g2, g3]
    bes = [be0, be1, be2, be3]
    Wn = [W1, W2, W3]

    out = None
    for i in range(4):
        s = _scatter_kernel(t, src, dst)
        br = bs[i].reshape(1, HID)
        gr = gs[i].reshape(1, HID)
        ber = bes[i].reshape(1, HID)
        if i < 3:
            t = pl.pallas_call(
                _layer_body,
                out_shape=jax.ShapeDtypeStruct((NP, HID), jnp.float32),
            )(s, t, degp, br, gr, ber, Wn[i])
        else:
            out = pl.pallas_call(
                _final_body,
                out_shape=jax.ShapeDtypeStruct((N, 1), jnp.float32),
            )(s, t, degp, br, gr, ber,
              hW0, hb0.reshape(1, HID), hW1.reshape(1, HID), hb1.reshape(1, 1))
    return out


# speedup vs baseline: 12.4968x; 1.1653x over previous
import functools

import jax
import jax.numpy as jnp
from jax import lax
from jax.experimental import pallas as pl
from jax.experimental.pallas import tpu as pltpu
from jax.experimental.pallas import tpu_sc as plsc

N = 10000
E = 320000
HID = 64
EPS = 1e-5

NC = 2
NS = 16
NW = NC * NS
NP = 10240
EP = 327680
CH = 128
NCH = EP // (NW * CH)
RPT = NP // NS

_mesh = plsc.VectorSubcoreMesh(
    core_axis_name="c", subcore_axis_name="s", num_cores=NC, num_subcores=NS)


@functools.partial(
    pl.kernel,
    out_type=jax.ShapeDtypeStruct((NC * NP, 16), jnp.float32),
    mesh=_mesh,
    scratch_types=[
        pltpu.VMEM((NCH, CH), jnp.int32),
        pltpu.VMEM((CH, 16), jnp.float32),
        pltpu.VMEM((RPT, 16), jnp.float32),
        pltpu.VMEM_SHARED((NP, 16), jnp.float32),
    ],
    compiler_params=pltpu.CompilerParams(use_tc_tiling_on_sc=False),
)
def _deg_kernel(dst_hbm, ones_hbm, zeros_hbm, out_hbm, dst_v, ones_v, stage_v, acc_s):
    c = lax.axis_index("c")
    s = lax.axis_index("s")
    wid = c * NS + s
    r0 = s * RPT
    pltpu.sync_copy(zeros_hbm.at[pl.ds(r0, RPT)], stage_v)
    pltpu.sync_copy(stage_v, acc_s.at[pl.ds(r0, RPT)])
    pltpu.sync_copy(ones_hbm, ones_v)
    pltpu.sync_copy(dst_hbm.at[pl.ds(wid * NCH, NCH)], dst_v)
    plsc.subcore_barrier()

    def body(j, carry):
        pltpu.sync_copy(ones_v, acc_s.at[dst_v.at[j]], add=True)
        return carry

    lax.fori_loop(0, NCH, body, 0)
    plsc.subcore_barrier()
    pltpu.sync_copy(acc_s.at[pl.ds(r0, RPT)], stage_v)
    pltpu.sync_copy(stage_v, out_hbm.at[pl.ds(c * NP + r0, RPT)])


NBUF = 8
GRP = NCH // NBUF


@functools.partial(
    pl.kernel,
    out_type=jax.ShapeDtypeStruct((NC * NP, HID), jnp.float32),
    mesh=_mesh,
    scratch_types=[
        pltpu.VMEM((NCH, CH), jnp.int32),
        pltpu.VMEM((NCH, CH), jnp.int32),
        pltpu.VMEM((NBUF, CH, HID), jnp.float32),
        pltpu.VMEM_SHARED((NP, HID), jnp.float32),
        pltpu.SemaphoreType.DMA((NBUF,)),
        pltpu.SemaphoreType.DMA((NBUF,)),
    ],
    compiler_params=pltpu.CompilerParams(use_tc_tiling_on_sc=False),
)
def _scatter_kernel(t_hbm, src_hbm, dst_hbm, out_hbm, src_v, dst_v, rows_v, acc_s, gsem, ssem):
    c = lax.axis_index("c")
    s = lax.axis_index("s")
    wid = c * NS + s
    r0 = s * RPT

    def initb(k, carry):
        pltpu.sync_copy(t_hbm.at[pl.ds(r0 + k * CH, CH)], rows_v.at[0])
        pltpu.sync_copy(rows_v.at[0], acc_s.at[pl.ds(r0 + k * CH, CH)])
        return carry

    lax.fori_loop(0, RPT // CH, initb, 0)
    pltpu.sync_copy(src_hbm.at[pl.ds(wid * NCH, NCH)], src_v)
    pltpu.sync_copy(dst_hbm.at[pl.ds(wid * NCH, NCH)], dst_v)
    plsc.subcore_barrier()

    for b in range(NBUF):
        pltpu.async_copy(t_hbm.at[src_v.at[b]], rows_v.at[b], gsem.at[b])

    def group(g, carry):
        jbase = g * NBUF
        for b in range(NBUF):
            j = jbase + b
            pltpu.make_async_copy(
                t_hbm.at[src_v.at[j]], rows_v.at[b], gsem.at[b]).wait()
            pltpu.async_copy(
                rows_v.at[b], acc_s.at[dst_v.at[j]], ssem.at[b], add=True)
        for b in range(NBUF):
            j = jbase + b
            pltpu.make_async_copy(
                rows_v.at[b], acc_s.at[dst_v.at[j]], ssem.at[b]).wait()

            @pl.when(g < GRP - 1)
            def _():
                pltpu.async_copy(
                    t_hbm.at[src_v.at[j + NBUF]], rows_v.at[b], gsem.at[b])
        return carry

    lax.fori_loop(0, GRP, group, 0)
    plsc.subcore_barrier()

    def outb(k, carry):
        pltpu.sync_copy(acc_s.at[pl.ds(r0 + k * CH, CH)], rows_v.at[0])
        pltpu.sync_copy(rows_v.at[0], out_hbm.at[pl.ds(c * NP + r0 + k * CH, CH)])
        return carry

    lax.fori_loop(0, RPT // CH, outb, 0)


def _dinv(degp_ref):
    deg = degp_ref[:N, :1] + degp_ref[NP:NP + N, :1] + 1.0
    return lax.rsqrt(deg)


def _pre_body(x_ref, w_ref, degp_ref, t_ref):
    dinv = _dinv(degp_ref)
    t_ref[:N, :] = jnp.dot(
        x_ref[...], w_ref[...], preferred_element_type=jnp.float32) * dinv
    t_ref[N:, :] = jnp.zeros((NP - N, HID), jnp.float32)


def _bn_relu(s_ref, t_ref, degp_ref, b_ref, g_ref, be_ref):
    dinv = _dinv(degp_ref)
    z = (s_ref[:N, :] + s_ref[NP:NP + N, :] - t_ref[:N, :]) * dinv + b_ref[...]
    mu = jnp.mean(z, axis=0, keepdims=True)
    d = z - mu
    var = jnp.mean(d * d, axis=0, keepdims=True)
    hn = jnp.maximum(d * lax.rsqrt(var + EPS) * g_ref[...] + be_ref[...], 0.0)
    return hn, dinv


def _layer_body(s_ref, t_ref, degp_ref, b_ref, g_ref, be_ref, w_ref, o_ref):
    hn, dinv = _bn_relu(s_ref, t_ref, degp_ref, b_ref, g_ref, be_ref)
    o_ref[:N, :] = jnp.dot(
        hn, w_ref[...], preferred_element_type=jnp.float32) * dinv
    o_ref[N:, :] = jnp.zeros((NP - N, HID), jnp.float32)


def _final_body(s_ref, t_ref, degp_ref, b_ref, g_ref, be_ref,
                hw0_ref, hb0_ref, hw1_ref, hb1_ref, o_ref):
    hn, _ = _bn_relu(s_ref, t_ref, degp_ref, b_ref, g_ref, be_ref)
    h2 = jnp.maximum(
        jnp.dot(hn, hw0_ref[...], preferred_element_type=jnp.float32) + hb0_ref[...], 0.0)
    o_ref[...] = jnp.sum(h2 * hw1_ref[...], axis=1, keepdims=True) + hb1_ref[...]


def kernel(x, edge_index, W0, b0, g0, be0, W1, b1, g1, be1, W2, b2, g2, be2,
           W3, b3, g3, be3, hW0, hb0, hW1, hb1):
    pad = jnp.full((EP - E,), N, dtype=jnp.int32)
    src = jnp.concatenate([edge_index[0], pad]).reshape(NW * NCH, CH)
    dst = jnp.concatenate([edge_index[1], pad]).reshape(NW * NCH, CH)
    ones16 = jnp.ones((CH, 16), jnp.float32)
    zeros16 = jnp.zeros((NP, 16), jnp.float32)

    degp = _deg_kernel(dst, ones16, zeros16)

    t = pl.pallas_call(
        _pre_body,
        out_shape=jax.ShapeDtypeStruct((NP, HID), jnp.float32),
    )(x, W0, degp)

    bs = [b0, b1, b2, b3]
    gs = [g0, g1, g2, g3]
    bes = [be0, be1, be2, be3]
    Wn = [W1, W2, W3]

    out = None
    for i in range(4):
        s = _scatter_kernel(t, src, dst)
        br = bs[i].reshape(1, HID)
        gr = gs[i].reshape(1, HID)
        ber = bes[i].reshape(1, HID)
        if i < 3:
            t = pl.pallas_call(
                _layer_body,
                out_shape=jax.ShapeDtypeStruct((NP, HID), jnp.float32),
            )(s, t, degp, br, gr, ber, Wn[i])
        else:
            out = pl.pallas_call(
                _final_body,
                out_shape=jax.ShapeDtypeStruct((N, 1), jnp.float32),
            )(s, t, degp, br, gr, ber,
              hW0, hb0.reshape(1, HID), hW1.reshape(1, HID), hb1.reshape(1, 1))
    return out


# speedup vs baseline: 14.5601x; 1.1651x over previous
import functools

import jax
import jax.numpy as jnp
from jax import lax
from jax.experimental import pallas as pl
from jax.experimental.pallas import tpu as pltpu
from jax.experimental.pallas import tpu_sc as plsc

N = 10000
E = 320000
HID = 64
HH = 32
EPS = 1e-5

NC = 2
NS = 16
NW = NC * NS
NP = 10240
EP = 327680
CH = 128
NCHUNKS = EP // CH

_mesh = plsc.VectorSubcoreMesh(
    core_axis_name="c", subcore_axis_name="s", num_cores=NC, num_subcores=NS)


NBUF = 8
NPH = NP // 2
JROWS = 128
ACCR = NPH + JROWS
NCHT = NCHUNKS // NS
GRP = NCHT // NBUF
APT = NPH // NS


@functools.partial(
    pl.kernel,
    out_type=jax.ShapeDtypeStruct((NP, HH), jnp.float32),
    mesh=_mesh,
    scratch_types=[
        pltpu.VMEM((NCHT, CH), jnp.int32),
        pltpu.VMEM((NCHT, CH), jnp.int32),
        pltpu.VMEM((NBUF, CH, HH), jnp.float32),
        pltpu.VMEM_SHARED((ACCR, HH), jnp.float32),
        pltpu.VMEM_SHARED((NP, HH), jnp.float32),
        pltpu.SemaphoreType.DMA((NBUF,)),
        pltpu.SemaphoreType.DMA((NBUF,)),
    ],
    compiler_params=pltpu.CompilerParams(use_tc_tiling_on_sc=False),
)
def _scatter_kernel(t_hbm, src_hbm, dst_hbm, out_hbm, src_v, dst_v, rows_v,
                    acc_s, ts_s, gsem, ssem):
    c = lax.axis_index("c")
    s = lax.axis_index("s")
    base = c * NPH

    r0 = s * (NP // NS)

    def initb(k, carry):
        pltpu.sync_copy(t_hbm.at[pl.ds(r0 + k * CH, CH)], rows_v.at[0])
        pltpu.sync_copy(rows_v.at[0], ts_s.at[pl.ds(r0 + k * CH, CH)])
        return carry

    lax.fori_loop(0, (NP // NS) // CH, initb, 0)

    a0 = s * APT

    def inita(k, carry):
        pltpu.sync_copy(
            t_hbm.at[pl.ds(base + a0 + k * 80, 80)], rows_v.at[1, pl.ds(0, 80)])
        pltpu.sync_copy(
            rows_v.at[1, pl.ds(0, 80)], acc_s.at[pl.ds(a0 + k * 80, 80)])
        return carry

    lax.fori_loop(0, APT // 80, inita, 0)

    j0 = s * NCHT
    pltpu.sync_copy(src_hbm.at[pl.ds(j0, NCHT)], src_v)
    pltpu.sync_copy(dst_hbm.at[pl.ds(j0, NCHT)], dst_v)

    def remap(j, carry):
        for k in range(CH // 16):
            v = dst_v[j, pl.ds(k * 16, 16)]
            local = v - base
            ok = (local >= 0) & (local < NPH)
            jnk = (v & (JROWS - 1)) + NPH
            dst_v[j, pl.ds(k * 16, 16)] = jnp.where(ok, local, jnk)
        return carry

    lax.fori_loop(0, NCHT, remap, 0)
    plsc.subcore_barrier()

    for b in range(NBUF):
        pltpu.async_copy(ts_s.at[src_v.at[b]], rows_v.at[b], gsem.at[b])

    def group(g, carry):
        jbase = g * NBUF
        for b in range(NBUF):
            j = jbase + b
            pltpu.make_async_copy(
                ts_s.at[src_v.at[j]], rows_v.at[b], gsem.at[b]).wait()
            pltpu.async_copy(
                rows_v.at[b], acc_s.at[dst_v.at[j]], ssem.at[b], add=True)
        for b in range(NBUF):
            j = jbase + b
            pltpu.make_async_copy(
                rows_v.at[b], acc_s.at[dst_v.at[j]], ssem.at[b]).wait()

            @pl.when(g < GRP - 1)
            def _():
                pltpu.async_copy(
                    ts_s.at[src_v.at[j + NBUF]], rows_v.at[b], gsem.at[b])
        return carry

    lax.fori_loop(0, GRP, group, 0)
    plsc.subcore_barrier()

    def outb(k, carry):
        pltpu.sync_copy(
            acc_s.at[pl.ds(a0 + k * 80, 80)], rows_v.at[0, pl.ds(0, 80)])
        pltpu.sync_copy(
            rows_v.at[0, pl.ds(0, 80)],
            out_hbm.at[pl.ds(base + a0 + k * 80, 80)])
        return carry

    lax.fori_loop(0, APT // 80, outb, 0)


def _pre_body(x_ref, w_ref, degs_ref, tlo_ref, thi_ref, dinv_ref):
    dinv = lax.rsqrt(degs_ref[:, :1])
    dinv_ref[...] = dinv
    t = jnp.dot(
        x_ref[...], w_ref[...], preferred_element_type=jnp.float32) * dinv[:N]
    tlo_ref[:N, :] = t[:, :HH]
    tlo_ref[N:, :] = jnp.zeros((NP - N, HH), jnp.float32)
    thi_ref[:N, :] = t[:, HH:]
    thi_ref[N:, :] = jnp.zeros((NP - N, HH), jnp.float32)


def _bn_relu(slo_ref, shi_ref, dinv_ref, b_ref, g_ref, be_ref):
    dinv = dinv_ref[:N]
    z = jnp.concatenate([slo_ref[:N, :], shi_ref[:N, :]], axis=1)
    z = z * dinv + b_ref[...]
    mu = jnp.mean(z, axis=0, keepdims=True)
    d = z - mu
    var = jnp.mean(d * d, axis=0, keepdims=True)
    hn = jnp.maximum(d * lax.rsqrt(var + EPS) * g_ref[...] + be_ref[...], 0.0)
    return hn, dinv


def _layer_body(slo_ref, shi_ref, dinv_ref, b_ref, g_ref, be_ref, w_ref,
                tlo_ref, thi_ref):
    hn, dinv = _bn_relu(slo_ref, shi_ref, dinv_ref, b_ref, g_ref, be_ref)
    t = jnp.dot(hn, w_ref[...], preferred_element_type=jnp.float32) * dinv
    tlo_ref[:N, :] = t[:, :HH]
    tlo_ref[N:, :] = jnp.zeros((NP - N, HH), jnp.float32)
    thi_ref[:N, :] = t[:, HH:]
    thi_ref[N:, :] = jnp.zeros((NP - N, HH), jnp.float32)


def _final_body(slo_ref, shi_ref, dinv_ref, b_ref, g_ref, be_ref,
                hw0_ref, hb0_ref, hw1_ref, hb1_ref, o_ref):
    hn, _ = _bn_relu(slo_ref, shi_ref, dinv_ref, b_ref, g_ref, be_ref)
    h2 = jnp.maximum(
        jnp.dot(hn, hw0_ref[...], preferred_element_type=jnp.float32)
        + hb0_ref[...], 0.0)
    o_ref[...] = jnp.sum(h2 * hw1_ref[...], axis=1, keepdims=True) + hb1_ref[...]


def kernel(x, edge_index, W0, b0, g0, be0, W1, b1, g1, be1, W2, b2, g2, be2,
           W3, b3, g3, be3, hW0, hb0, hW1, hb1):
    pad = jnp.full((EP - E,), N, dtype=jnp.int32)
    src = jnp.concatenate([edge_index[0], pad]).reshape(NCHUNKS, CH)
    dst = jnp.concatenate([edge_index[1], pad]).reshape(NCHUNKS, CH)

    degs = _scatter_kernel(jnp.ones((NP, HH), jnp.float32), src, dst)

    tlo, thi, dinv = pl.pallas_call(
        _pre_body,
        out_shape=(jax.ShapeDtypeStruct((NP, HH), jnp.float32),
                   jax.ShapeDtypeStruct((NP, HH), jnp.float32),
                   jax.ShapeDtypeStruct((NP, 1), jnp.float32)),
    )(x, W0, degs)

    bs = [b0, b1, b2, b3]
    gs = [g0, g1, g2, g3]
    bes = [be0, be1, be2, be3]
    Wn = [W1, W2, W3]

    out = None
    for i in range(4):
        slo = _scatter_kernel(tlo, src, dst)
        shi = _scatter_kernel(thi, src, dst)
        br = bs[i].reshape(1, HID)
        gr = gs[i].reshape(1, HID)
        ber = bes[i].reshape(1, HID)
        if i < 3:
            tlo, thi = pl.pallas_call(
                _layer_body,
                out_shape=(jax.ShapeDtypeStruct((NP, HH), jnp.float32),
                           jax.ShapeDtypeStruct((NP, HH), jnp.float32)),
            )(slo, shi, dinv, br, gr, ber, Wn[i])
        else:
            out = pl.pallas_call(
                _final_body,
                out_shape=jax.ShapeDtypeStruct((N, 1), jnp.float32),
            )(slo, shi, dinv, br, gr, ber,
              hW0, hb0.reshape(1, HID), hW1.reshape(1, HID), hb1.reshape(1, 1))
    return out
